# Initial kernel scaffold; baseline (speedup 1.0000x reference)
import functools

import jax
import jax.numpy as jnp
from jax import lax
from jax.experimental import pallas as pl
from jax.experimental.pallas import tpu as pltpu
from jax.experimental.pallas import tpu_sc as plsc

N = 10000
E = 160000
D = 300
DP = 320
DH = DP // 2
L = 5
NGRAPH = 64
VPAD = 128

NC = 2
NS = 16
EPS = E // NS
CH = 80
NCHUNK = EPS // CH
RSTG = 125
NPS = N // NS


def _edge_body(h_hbm, ee_hbm, src_hbm, dst_hbm, out_hbm,
               src_v, dst_v, gat_v, msg_v, stg_v, agg_sh, sem):
    c = lax.axis_index("c")
    s = lax.axis_index("s")

    r0 = s * NPS
    for k in range(NPS // RSTG):
        rk = r0 + k * RSTG
        pltpu.sync_copy(h_hbm.at[pl.ds(c * N + rk, RSTG)], stg_v)
        pltpu.sync_copy(stg_v, agg_sh.at[pl.ds(rk, RSTG)])
    plsc.subcore_barrier()

    def chunk(i, _):
        e0 = s * EPS + i * CH
        pltpu.sync_copy(src_hbm.at[pl.ds(e0, CH)], src_v)
        pltpu.sync_copy(dst_hbm.at[pl.ds(e0, CH)], dst_v)
        pltpu.sync_copy(ee_hbm.at[pl.ds(c * E + e0, CH)], msg_v)
        for j in range(CH // 16):
            sl = pl.ds(j * 16, 16)
            src_v[sl] = src_v[sl] + c * N
        pltpu.async_copy(h_hbm.at[src_v], gat_v, sem).wait()

        def row(r, _):
            for j in range(DH // 16):
                sl = (r, pl.ds(j * 16, 16))
                msg_v[sl] = jnp.maximum(msg_v[sl] + gat_v[sl], 0.0)
            return 0

        lax.fori_loop(0, CH, row, 0)
        pltpu.sync_copy(msg_v, agg_sh.at[dst_v], add=True)
        return 0

    lax.fori_loop(0, NCHUNK, chunk, 0)
    plsc.subcore_barrier()

    for k in range(NPS // RSTG):
        rk = r0 + k * RSTG
        pltpu.sync_copy(agg_sh.at[pl.ds(rk, RSTG)], stg_v)
        pltpu.sync_copy(stg_v, out_hbm.at[pl.ds(c * N + rk, RSTG)])


_edge_sc = functools.partial(
    pl.kernel,
    out_type=jax.ShapeDtypeStruct((2 * N, DH), jnp.float32),
    mesh=plsc.VectorSubcoreMesh(core_axis_name="c", subcore_axis_name="s",
                                num_cores=NC, num_subcores=NS),
    scratch_types=[
        pltpu.VMEM((CH,), jnp.int32),
        pltpu.VMEM((CH,), jnp.int32),
        pltpu.VMEM((CH, DH), jnp.float32),
        pltpu.VMEM((CH, DH), jnp.float32),
        pltpu.VMEM((RSTG, DH), jnp.float32),
        pltpu.VMEM_SHARED((N, DH), jnp.float32),
        pltpu.SemaphoreType.DMA,
    ],
)(_edge_body)


def _ee_mlp_body(ea_ref, w1_ref, b1_ref, w2_ref, b2_ref, out_ref):
    ea = ea_ref[:]
    t = (ea[:, 0:1] * w1_ref[0:1, :] + ea[:, 1:2] * w1_ref[1:2, :]
         + ea[:, 2:3] * w1_ref[2:3, :] + b1_ref[:])
    t = jnp.maximum(t, 0.0)
    eeb = jnp.dot(t, w2_ref[:], preferred_element_type=jnp.float32) + b2_ref[:]
    out_ref[0] = eeb[:, :DH]
    out_ref[1] = eeb[:, DH:]


def _ee_mlp(ea, w1, b1, w2, b2):
    BE = 2000
    return pl.pallas_call(
        _ee_mlp_body,
        grid=(E // BE,),
        in_specs=[
            pl.BlockSpec((BE, 3), lambda i: (i, 0)),
            pl.BlockSpec((3, DP), lambda i: (0, 0)),
            pl.BlockSpec((1, DP), lambda i: (0, 0)),
            pl.BlockSpec((DP, DP), lambda i: (0, 0)),
            pl.BlockSpec((1, DP), lambda i: (0, 0)),
        ],
        out_specs=pl.BlockSpec((2, BE, DH), lambda i: (0, i, 0)),
        out_shape=jax.ShapeDtypeStruct((2, E, DH), jnp.float32),
        compiler_params=pltpu.CompilerParams(
            dimension_semantics=("arbitrary",)),
    )(ea, w1, b1, w2, b2)


def _node_init_body(z_ref, ch_ref, fc_ref, at_ref, w1_ref, b1_ref,
                    w2_ref, b2_ref, out_ref):
    zcol = z_ref[:]
    onehot = jnp.where(
        lax.broadcasted_iota(jnp.int32, (N, VPAD), 1) == zcol, 1.0, 0.0)
    x = jnp.dot(onehot, at_ref[:], preferred_element_type=jnp.float32)
    t = (ch_ref[:] * w1_ref[0:1, :] + fc_ref[:] * w1_ref[1:2, :] + b1_ref[:])
    t = jnp.maximum(t, 0.0)
    x = x + jnp.dot(t, w2_ref[:], preferred_element_type=jnp.float32) + b2_ref[:]
    out_ref[0] = x[:, :DH]
    out_ref[1] = x[:, DH:]


def _node_init(z2, ch2, fc2, at_p, w1, b1, w2, b2):
    return pl.pallas_call(
        _node_init_body,
        out_shape=jax.ShapeDtypeStruct((2, N, DH), jnp.float32),
    )(z2, ch2, fc2, at_p, w1, b1, w2, b2)


def _mlp_bn_body(hin_ref, w1_ref, b1_ref, w2_ref, b2_ref, g_ref, bt_ref,
                 out_ref):
    t = (jnp.dot(hin_ref[0], w1_ref[:DH, :], preferred_element_type=jnp.float32)
         + jnp.dot(hin_ref[1], w1_ref[DH:, :], preferred_element_type=jnp.float32)
         + b1_ref[:])
    t = jnp.maximum(t, 0.0)
    h2 = jnp.dot(t, w2_ref[:], preferred_element_type=jnp.float32) + b2_ref[:]
    mean = jnp.mean(h2, axis=0, keepdims=True)
    var = jnp.mean((h2 - mean) ** 2, axis=0, keepdims=True)
    hn = (h2 - mean) * lax.rsqrt(var + 1e-5) * g_ref[:] + bt_ref[:]
    hn = jnp.maximum(hn, 0.0)
    out_ref[0] = hn[:, :DH]
    out_ref[1] = hn[:, DH:]


def _mlp_bn(hin, w1, b1, w2, b2, g, bt):
    return pl.pallas_call(
        _mlp_bn_body,
        out_shape=jax.ShapeDtypeStruct((2, N, DH), jnp.float32),
    )(hin, w1, b1, w2, b2, g, bt)


def _pool_body(h_ref, b_ref, w_ref, bias_ref, out_ref):
    h = jnp.concatenate([h_ref[0], h_ref[1]], axis=1)
    onehot_t = jnp.where(
        lax.broadcasted_iota(jnp.int32, (NGRAPH, N), 0) == b_ref[:], 1.0, 0.0)
    sums = jnp.dot(onehot_t, h, preferred_element_type=jnp.float32)
    counts = jnp.sum(onehot_t, axis=1, keepdims=True)
    pooled = jnp.where(counts > 0, sums / jnp.maximum(counts, 1.0), 0.0)
    out_ref[:] = (jnp.dot(pooled, w_ref[:], preferred_element_type=jnp.float32)
                  + bias_ref[:])


def _pool(h, b2, w, bias):
    return pl.pallas_call(
        _pool_body,
        out_shape=jax.ShapeDtypeStruct((NGRAPH, DP), jnp.float32),
    )(h, b2, w, bias)


def _padc(x, cols=DP):
    return jnp.pad(x, ((0, 0), (0, cols - x.shape[1])))


def kernel(z, chirality, formal_charge, edge_index, edge_attr, batch,
           atom_table, nap_w1, nap_b1, nap_w2, nap_b2, ee_w1, ee_b1,
           ee_w2, ee_b2, mlp_w1, mlp_b1, mlp_w2, mlp_b2, bn_gamma, bn_beta,
           pool_w, pool_b):
    f32 = jnp.float32
    at_p = jnp.pad(atom_table.astype(f32), ((0, VPAD - atom_table.shape[0]),
                                            (0, DP - D)))
    nap_w1p = _padc(nap_w1)
    nap_b1p = _padc(nap_b1[None, :])
    nap_w2p = _padc(jnp.pad(nap_w2, ((0, DP - D), (0, 0))))
    nap_b2p = _padc(nap_b2[None, :])
    ee_w1p = _padc(ee_w1)
    ee_b1p = _padc(ee_b1[None, :])
    ee_w2p = _padc(jnp.pad(ee_w2, ((0, DP - D), (0, 0))))
    ee_b2p = _padc(ee_b2[None, :])
    w1p = jnp.pad(mlp_w1, ((0, 0), (0, DP - D), (0, DP - D)))
    w2p = jnp.pad(mlp_w2, ((0, 0), (0, DP - D), (0, DP - D)))
    b1p = jnp.pad(mlp_b1, ((0, 0), (0, DP - D)))[:, None, :]
    b2p = jnp.pad(mlp_b2, ((0, 0), (0, DP - D)))[:, None, :]
    gp = jnp.pad(bn_gamma, ((0, 0), (0, DP - D)))[:, None, :]
    btp = jnp.pad(bn_beta, ((0, 0), (0, DP - D)))[:, None, :]
    pool_wp = _padc(jnp.pad(pool_w, ((0, DP - D), (0, 0))))
    pool_bp = _padc(pool_b[None, :])

    src = edge_index[0].astype(jnp.int32)
    dst = edge_index[1].astype(jnp.int32)
    z2 = z.astype(jnp.int32)[:, None]
    ch2 = chirality.astype(f32)[:, None]
    fc2 = formal_charge.astype(f32)[:, None]
    b2d = batch.astype(jnp.int32)[None, :]

    ee = _ee_mlp(edge_attr.astype(f32), ee_w1p, ee_b1p, ee_w2p, ee_b2p)
    h = _node_init(z2, ch2, fc2, at_p, nap_w1p, nap_b1p, nap_w2p, nap_b2p)
    ee_flat = ee.reshape(2 * E, DH)
    for i in range(L):
        hin = _edge_sc(h.reshape(2 * N, DH), ee_flat, src, dst)
        h = _mlp_bn(hin.reshape(2, N, DH), w1p[i], b1p[i], w2p[i], b2p[i],
                    gp[i], btp[i])
    out = _pool(h, b2d, pool_wp, pool_bp)
    return out[:, :D]


# speedup vs baseline: 2.1689x; 2.1689x over previous
import functools

import jax
import jax.numpy as jnp
from jax import lax
from jax.experimental import pallas as pl
from jax.experimental.pallas import tpu as pltpu
from jax.experimental.pallas import tpu_sc as plsc

N = 10000
E = 160000
D = 300
DP = 320
DH = DP // 2
L = 5
NGRAPH = 64
VPAD = 128

NC = 2
NS = 16
EPS = E // NS
CH = 80
NCHUNK = EPS // CH
RSTG = CH
NRB = N // RSTG


def _edge_body(h_hbm, ee_hbm, src_hbm, dst_hbm, out_hbm,
               src_v, dst_v, gat_v, msg_v, agg_sh, sem):
    c = lax.axis_index("c")
    s = lax.axis_index("s")

    def seed(k, _):
        b = s + NS * k

        @pl.when(b < NRB)
        def _():
            rk = b * RSTG
            pltpu.sync_copy(h_hbm.at[pl.ds(c * N + rk, RSTG)], msg_v)
            pltpu.sync_copy(msg_v, agg_sh.at[pl.ds(rk, RSTG)])

        return 0

    lax.fori_loop(0, (NRB + NS - 1) // NS, seed, 0)
    plsc.subcore_barrier()

    def chunk(i, _):
        e0 = s * EPS + i * CH
        pltpu.sync_copy(src_hbm.at[pl.ds(e0, CH)], src_v)
        pltpu.sync_copy(dst_hbm.at[pl.ds(e0, CH)], dst_v)
        pltpu.sync_copy(ee_hbm.at[pl.ds(c * E + e0, CH)], msg_v)
        for j in range(CH // 16):
            sl = pl.ds(j * 16, 16)
            src_v[sl] = src_v[sl] + c * N
        pltpu.async_copy(h_hbm.at[src_v], gat_v, sem).wait()

        def row(r, _):
            for j in range(DH // 16):
                sl = (r, pl.ds(j * 16, 16))
                msg_v[sl] = jnp.maximum(msg_v[sl] + gat_v[sl], 0.0)
            return 0

        lax.fori_loop(0, CH, row, 0)
        pltpu.sync_copy(msg_v, agg_sh.at[dst_v], add=True)
        return 0

    lax.fori_loop(0, NCHUNK, chunk, 0)
    plsc.subcore_barrier()

    def drain(k, _):
        b = s + NS * k

        @pl.when(b < NRB)
        def _():
            rk = b * RSTG
            pltpu.sync_copy(agg_sh.at[pl.ds(rk, RSTG)], msg_v)
            pltpu.sync_copy(msg_v, out_hbm.at[pl.ds(c * N + rk, RSTG)])

        return 0

    lax.fori_loop(0, (NRB + NS - 1) // NS, drain, 0)


_edge_sc = functools.partial(
    pl.kernel,
    out_type=jax.ShapeDtypeStruct((2 * N, DH), jnp.float32),
    mesh=plsc.VectorSubcoreMesh(core_axis_name="c", subcore_axis_name="s",
                                num_cores=NC, num_subcores=NS),
    scratch_types=[
        pltpu.VMEM((CH,), jnp.int32),
        pltpu.VMEM((CH,), jnp.int32),
        pltpu.VMEM((CH, DH), jnp.float32),
        pltpu.VMEM((CH, DH), jnp.float32),
        pltpu.VMEM_SHARED((N, DH), jnp.float32),
        pltpu.SemaphoreType.DMA,
    ],
    compiler_params=pltpu.CompilerParams(use_tc_tiling_on_sc=False),
)(_edge_body)


def _ee_mlp_body(ea_ref, w1_ref, b1_ref, w2_ref, b2_ref, out_ref):
    ea = ea_ref[:]
    t = (ea[:, 0:1] * w1_ref[0:1, :] + ea[:, 1:2] * w1_ref[1:2, :]
         + ea[:, 2:3] * w1_ref[2:3, :] + b1_ref[:])
    t = jnp.maximum(t, 0.0)
    eeb = jnp.dot(t, w2_ref[:], preferred_element_type=jnp.float32) + b2_ref[:]
    out_ref[0] = eeb[:, :DH]
    out_ref[1] = eeb[:, DH:]


def _ee_mlp(ea, w1, b1, w2, b2):
    BE = 2000
    return pl.pallas_call(
        _ee_mlp_body,
        grid=(E // BE,),
        in_specs=[
            pl.BlockSpec((BE, 3), lambda i: (i, 0)),
            pl.BlockSpec((3, DP), lambda i: (0, 0)),
            pl.BlockSpec((1, DP), lambda i: (0, 0)),
            pl.BlockSpec((DP, DP), lambda i: (0, 0)),
            pl.BlockSpec((1, DP), lambda i: (0, 0)),
        ],
        out_specs=pl.BlockSpec((2, BE, DH), lambda i: (0, i, 0)),
        out_shape=jax.ShapeDtypeStruct((2, E, DH), jnp.float32),
        compiler_params=pltpu.CompilerParams(
            dimension_semantics=("arbitrary",)),
    )(ea, w1, b1, w2, b2)


BN_ROWS = 2000
NBLK = N // BN_ROWS


def _node_init_body(z_ref, ch_ref, fc_ref, at_ref, w1_ref, b1_ref,
                    w2_ref, b2_ref, out_ref):
    zcol = z_ref[:]
    onehot = jnp.where(
        lax.broadcasted_iota(jnp.int32, (BN_ROWS, VPAD), 1) == zcol, 1.0, 0.0)
    x = jnp.dot(onehot, at_ref[:], preferred_element_type=jnp.float32)
    t = (ch_ref[:] * w1_ref[0:1, :] + fc_ref[:] * w1_ref[1:2, :] + b1_ref[:])
    t = jnp.maximum(t, 0.0)
    x = x + jnp.dot(t, w2_ref[:], preferred_element_type=jnp.float32) + b2_ref[:]
    out_ref[0] = x[:, :DH]
    out_ref[1] = x[:, DH:]


def _node_init(z2, ch2, fc2, at_p, w1, b1, w2, b2):
    return pl.pallas_call(
        _node_init_body,
        grid=(NBLK,),
        in_specs=[
            pl.BlockSpec((BN_ROWS, 1), lambda i: (i, 0)),
            pl.BlockSpec((BN_ROWS, 1), lambda i: (i, 0)),
            pl.BlockSpec((BN_ROWS, 1), lambda i: (i, 0)),
            pl.BlockSpec((VPAD, DP), lambda i: (0, 0)),
            pl.BlockSpec((2, DP), lambda i: (0, 0)),
            pl.BlockSpec((1, DP), lambda i: (0, 0)),
            pl.BlockSpec((DP, DP), lambda i: (0, 0)),
            pl.BlockSpec((1, DP), lambda i: (0, 0)),
        ],
        out_specs=pl.BlockSpec((2, BN_ROWS, DH), lambda i: (0, i, 0)),
        out_shape=jax.ShapeDtypeStruct((2, N, DH), jnp.float32),
        compiler_params=pltpu.CompilerParams(
            dimension_semantics=("arbitrary",)),
    )(z2, ch2, fc2, at_p, w1, b1, w2, b2)


def _mlp1_body(hin_ref, w1_ref, b1_ref, w2_ref, b2_ref, h2_ref, st_ref,
               acc_ref):
    i = pl.program_id(0)
    t = (jnp.dot(hin_ref[0], w1_ref[:DH, :], preferred_element_type=jnp.float32)
         + jnp.dot(hin_ref[1], w1_ref[DH:, :], preferred_element_type=jnp.float32)
         + b1_ref[:])
    t = jnp.maximum(t, 0.0)
    h2 = jnp.dot(t, w2_ref[:], preferred_element_type=jnp.float32) + b2_ref[:]
    h2_ref[0] = h2[:, :DH]
    h2_ref[1] = h2[:, DH:]
    ssum = jnp.sum(h2, axis=0, keepdims=True)
    ssq = jnp.sum(h2 * h2, axis=0, keepdims=True)
    blk = jnp.concatenate([ssum, ssq], axis=0)

    @pl.when(i == 0)
    def _():
        acc_ref[:] = blk

    @pl.when(i > 0)
    def _():
        acc_ref[:] = acc_ref[:] + blk

    @pl.when(i == NBLK - 1)
    def _():
        st_ref[:] = acc_ref[:]


def _mlp2_body(h2_ref, st_ref, g_ref, bt_ref, out_ref):
    m = st_ref[0:1, :] * (1.0 / N)
    v = st_ref[1:2, :] * (1.0 / N) - m * m
    inv = lax.rsqrt(v + 1e-5)
    for half in range(2):
        lo, hi = half * DH, (half + 1) * DH
        hn = ((h2_ref[half] - m[:, lo:hi]) * inv[:, lo:hi] * g_ref[:, lo:hi]
              + bt_ref[:, lo:hi])
        out_ref[half] = jnp.maximum(hn, 0.0)


def _mlp_bn(hin, w1, b1, w2, b2, g, bt):
    h2, st = pl.pallas_call(
        _mlp1_body,
        grid=(NBLK,),
        in_specs=[
            pl.BlockSpec((2, BN_ROWS, DH), lambda i: (0, i, 0)),
            pl.BlockSpec((DP, DP), lambda i: (0, 0)),
            pl.BlockSpec((1, DP), lambda i: (0, 0)),
            pl.BlockSpec((DP, DP), lambda i: (0, 0)),
            pl.BlockSpec((1, DP), lambda i: (0, 0)),
        ],
        out_specs=[
            pl.BlockSpec((2, BN_ROWS, DH), lambda i: (0, i, 0)),
            pl.BlockSpec((2, DP), lambda i: (0, 0)),
        ],
        out_shape=[
            jax.ShapeDtypeStruct((2, N, DH), jnp.float32),
            jax.ShapeDtypeStruct((2, DP), jnp.float32),
        ],
        scratch_shapes=[pltpu.VMEM((2, DP), jnp.float32)],
        compiler_params=pltpu.CompilerParams(
            dimension_semantics=("arbitrary",)),
    )(hin, w1, b1, w2, b2)
    return pl.pallas_call(
        _mlp2_body,
        grid=(NBLK,),
        in_specs=[
            pl.BlockSpec((2, BN_ROWS, DH), lambda i: (0, i, 0)),
            pl.BlockSpec((2, DP), lambda i: (0, 0)),
            pl.BlockSpec((1, DP), lambda i: (0, 0)),
            pl.BlockSpec((1, DP), lambda i: (0, 0)),
        ],
        out_specs=pl.BlockSpec((2, BN_ROWS, DH), lambda i: (0, i, 0)),
        out_shape=jax.ShapeDtypeStruct((2, N, DH), jnp.float32),
        compiler_params=pltpu.CompilerParams(
            dimension_semantics=("arbitrary",)),
    )(h2, st, g, bt)


def _pool_body(h_ref, b_ref, w_ref, bias_ref, out_ref):
    h = jnp.concatenate([h_ref[0], h_ref[1]], axis=1)
    onehot_t = jnp.where(
        lax.broadcasted_iota(jnp.int32, (NGRAPH, N), 0) == b_ref[:], 1.0, 0.0)
    sums = jnp.dot(onehot_t, h, preferred_element_type=jnp.float32)
    counts = jnp.sum(onehot_t, axis=1, keepdims=True)
    pooled = jnp.where(counts > 0, sums / jnp.maximum(counts, 1.0), 0.0)
    out_ref[:] = (jnp.dot(pooled, w_ref[:], preferred_element_type=jnp.float32)
                  + bias_ref[:])


def _pool(h, b2, w, bias):
    return pl.pallas_call(
        _pool_body,
        out_shape=jax.ShapeDtypeStruct((NGRAPH, DP), jnp.float32),
    )(h, b2, w, bias)


def _padc(x, cols=DP):
    return jnp.pad(x, ((0, 0), (0, cols - x.shape[1])))


def kernel(z, chirality, formal_charge, edge_index, edge_attr, batch,
           atom_table, nap_w1, nap_b1, nap_w2, nap_b2, ee_w1, ee_b1,
           ee_w2, ee_b2, mlp_w1, mlp_b1, mlp_w2, mlp_b2, bn_gamma, bn_beta,
           pool_w, pool_b):
    f32 = jnp.float32
    at_p = jnp.pad(atom_table.astype(f32), ((0, VPAD - atom_table.shape[0]),
                                            (0, DP - D)))
    nap_w1p = _padc(nap_w1)
    nap_b1p = _padc(nap_b1[None, :])
    nap_w2p = _padc(jnp.pad(nap_w2, ((0, DP - D), (0, 0))))
    nap_b2p = _padc(nap_b2[None, :])
    ee_w1p = _padc(ee_w1)
    ee_b1p = _padc(ee_b1[None, :])
    ee_w2p = _padc(jnp.pad(ee_w2, ((0, DP - D), (0, 0))))
    ee_b2p = _padc(ee_b2[None, :])
    w1p = jnp.pad(mlp_w1, ((0, 0), (0, DP - D), (0, DP - D)))
    w2p = jnp.pad(mlp_w2, ((0, 0), (0, DP - D), (0, DP - D)))
    b1p = jnp.pad(mlp_b1, ((0, 0), (0, DP - D)))[:, None, :]
    b2p = jnp.pad(mlp_b2, ((0, 0), (0, DP - D)))[:, None, :]
    gp = jnp.pad(bn_gamma, ((0, 0), (0, DP - D)))[:, None, :]
    btp = jnp.pad(bn_beta, ((0, 0), (0, DP - D)))[:, None, :]
    pool_wp = _padc(jnp.pad(pool_w, ((0, DP - D), (0, 0))))
    pool_bp = _padc(pool_b[None, :])

    src = edge_index[0].astype(jnp.int32)
    dst = edge_index[1].astype(jnp.int32)
    z2 = z.astype(jnp.int32)[:, None]
    ch2 = chirality.astype(f32)[:, None]
    fc2 = formal_charge.astype(f32)[:, None]
    b2d = batch.astype(jnp.int32)[None, :]

    ee = _ee_mlp(edge_attr.astype(f32), ee_w1p, ee_b1p, ee_w2p, ee_b2p)
    h = _node_init(z2, ch2, fc2, at_p, nap_w1p, nap_b1p, nap_w2p, nap_b2p)
    ee_flat = ee.reshape(2 * E, DH)
    for i in range(L):
        hin = _edge_sc(h.reshape(2 * N, DH), ee_flat, src, dst)
        h = _mlp_bn(hin.reshape(2, N, DH), w1p[i], b1p[i], w2p[i], b2p[i],
                    gp[i], btp[i])
    out = _pool(h, b2d, pool_wp, pool_bp)
    return out[:, :D]


# speedup vs baseline: 3.6019x; 1.6607x over previous
import functools

import jax
import jax.numpy as jnp
from jax import lax
from jax.experimental import pallas as pl
from jax.experimental.pallas import tpu as pltpu
from jax.experimental.pallas import tpu_sc as plsc

N = 10000
E = 160000
D = 300
DP = 320
DH = DP // 2
L = 5
NGRAPH = 64
VPAD = 128

NC = 2
NS = 16
EPS = E // NS
CH = 40
NCHUNK = EPS // CH
RSTG = CH
NRB = N // RSTG


def _edge_body(h_hbm, ee_hbm, srcx_hbm, dst_hbm, out_hbm,
               src0, src1, dst0, dst1, gat0, gat1, msg0, msg1, agg_sh,
               sem_is0, sem_is1, sem_id0, sem_id1,
               sem_ld0, sem_ld1, sem_sc0, sem_sc1):
    src_v = [src0, src1]
    dst_v = [dst0, dst1]
    gat_v = [gat0, gat1]
    msg_v = [msg0, msg1]
    sem_is = [sem_is0, sem_is1]
    sem_id = [sem_id0, sem_id1]
    sem_ld = [sem_ld0, sem_ld1]
    sem_sc = [sem_sc0, sem_sc1]
    c = lax.axis_index("c")
    s = lax.axis_index("s")

    def seed(k, _):
        b = s + NS * k

        @pl.when(b < NRB)
        def _():
            rk = b * RSTG
            pltpu.sync_copy(h_hbm.at[pl.ds(c * N + rk, RSTG)], msg_v[0])
            pltpu.sync_copy(msg_v[0], agg_sh.at[pl.ds(rk, RSTG)])

        return 0

    lax.fori_loop(0, (NRB + NS - 1) // NS, seed, 0)
    plsc.subcore_barrier()

    def esl(i):
        return pl.ds(s * EPS + i * CH, CH)

    def xsl(i):
        return pl.ds(c * E + s * EPS + i * CH, CH)

    def eesl(i):
        return pl.ds(c * E + s * EPS + i * CH, CH)

    def step(b, i):
        nb = 1 - b

        @pl.when(i > 0)
        def _():
            pltpu.make_async_copy(msg_v[nb], agg_sh.at[dst_v[nb]],
                                  sem_sc[nb]).wait()

        @pl.when(i + 1 < NCHUNK)
        def _():
            pltpu.async_copy(dst_hbm.at[esl(i + 1)], dst_v[nb], sem_id[nb])
            pltpu.make_async_copy(srcx_hbm.at[xsl(i + 1)], src_v[nb],
                                  sem_is[nb]).wait()
            pltpu.async_copy(ee_hbm.at[eesl(i + 1)], msg_v[nb], sem_ld[nb])
            pltpu.async_copy(h_hbm.at[src_v[nb]], gat_v[nb], sem_ld[nb])

        pltpu.make_async_copy(h_hbm.at[src_v[b]], gat_v[b], sem_ld[b]).wait()
        pltpu.make_async_copy(ee_hbm.at[pl.ds(0, CH)], msg_v[b],
                              sem_ld[b]).wait()

        @pl.when(i + 2 < NCHUNK)
        def _():
            pltpu.async_copy(srcx_hbm.at[xsl(i + 2)], src_v[b], sem_is[b])

        mv, gv = msg_v[b], gat_v[b]

        def row(r, _):
            for j in range(DH // 16):
                sl = (r, pl.ds(j * 16, 16))
                mv[sl] = jnp.maximum(mv[sl] + gv[sl], 0.0)
            return 0

        lax.fori_loop(0, CH, row, 0)
        pltpu.make_async_copy(dst_hbm.at[esl(i)], dst_v[b], sem_id[b]).wait()
        pltpu.async_copy(msg_v[b], agg_sh.at[dst_v[b]], sem_sc[b], add=True)

    pltpu.sync_copy(srcx_hbm.at[xsl(0)], src_v[0])
    pltpu.async_copy(dst_hbm.at[esl(0)], dst_v[0], sem_id[0])
    pltpu.async_copy(ee_hbm.at[eesl(0)], msg_v[0], sem_ld[0])
    pltpu.async_copy(h_hbm.at[src_v[0]], gat_v[0], sem_ld[0])
    pltpu.async_copy(srcx_hbm.at[xsl(1)], src_v[1], sem_is[1])

    def pair(j, _):
        step(0, 2 * j)
        step(1, 2 * j + 1)
        return 0

    lax.fori_loop(0, NCHUNK // 2, pair, 0)
    pltpu.make_async_copy(msg_v[1], agg_sh.at[dst_v[1]], sem_sc[1]).wait()
    plsc.subcore_barrier()

    def drain(k, _):
        b = s + NS * k

        @pl.when(b < NRB)
        def _():
            rk = b * RSTG
            pltpu.sync_copy(agg_sh.at[pl.ds(rk, RSTG)], msg_v[0])
            pltpu.sync_copy(msg_v[0], out_hbm.at[pl.ds(c * N + rk, RSTG)])

        return 0

    lax.fori_loop(0, (NRB + NS - 1) // NS, drain, 0)


_edge_sc = functools.partial(
    pl.kernel,
    out_type=jax.ShapeDtypeStruct((2 * N, DH), jnp.float32),
    mesh=plsc.VectorSubcoreMesh(core_axis_name="c", subcore_axis_name="s",
                                num_cores=NC, num_subcores=NS),
    scratch_types=(
        [pltpu.VMEM((CH,), jnp.int32)] * 4
        + [pltpu.VMEM((CH, DH), jnp.float32)] * 4
        + [pltpu.VMEM_SHARED((N, DH), jnp.float32)]
        + [pltpu.SemaphoreType.DMA] * 8
    ),
    compiler_params=pltpu.CompilerParams(use_tc_tiling_on_sc=False),
)(_edge_body)


def _ee_mlp_body(ea_ref, w1_ref, b1_ref, w2_ref, b2_ref, out_ref):
    ea = ea_ref[:]
    t = (ea[:, 0:1] * w1_ref[0:1, :] + ea[:, 1:2] * w1_ref[1:2, :]
         + ea[:, 2:3] * w1_ref[2:3, :] + b1_ref[:])
    t = jnp.maximum(t, 0.0)
    eeb = jnp.dot(t, w2_ref[:], preferred_element_type=jnp.float32) + b2_ref[:]
    out_ref[0] = eeb[:, :DH]
    out_ref[1] = eeb[:, DH:]


def _ee_mlp(ea, w1, b1, w2, b2):
    BE = 2000
    return pl.pallas_call(
        _ee_mlp_body,
        grid=(E // BE,),
        in_specs=[
            pl.BlockSpec((BE, 3), lambda i: (i, 0)),
            pl.BlockSpec((3, DP), lambda i: (0, 0)),
            pl.BlockSpec((1, DP), lambda i: (0, 0)),
            pl.BlockSpec((DP, DP), lambda i: (0, 0)),
            pl.BlockSpec((1, DP), lambda i: (0, 0)),
        ],
        out_specs=pl.BlockSpec((2, BE, DH), lambda i: (0, i, 0)),
        out_shape=jax.ShapeDtypeStruct((2, E, DH), jnp.float32),
        compiler_params=pltpu.CompilerParams(
            dimension_semantics=("arbitrary",)),
    )(ea, w1, b1, w2, b2)


BN_ROWS = 2000
NBLK = N // BN_ROWS


def _node_init_body(z_ref, ch_ref, fc_ref, at_ref, w1_ref, b1_ref,
                    w2_ref, b2_ref, out_ref):
    zcol = z_ref[:]
    onehot = jnp.where(
        lax.broadcasted_iota(jnp.int32, (BN_ROWS, VPAD), 1) == zcol, 1.0, 0.0)
    x = jnp.dot(onehot, at_ref[:], preferred_element_type=jnp.float32)
    t = (ch_ref[:] * w1_ref[0:1, :] + fc_ref[:] * w1_ref[1:2, :] + b1_ref[:])
    t = jnp.maximum(t, 0.0)
    x = x + jnp.dot(t, w2_ref[:], preferred_element_type=jnp.float32) + b2_ref[:]
    out_ref[0] = x[:, :DH]
    out_ref[1] = x[:, DH:]


def _node_init(z2, ch2, fc2, at_p, w1, b1, w2, b2):
    return pl.pallas_call(
        _node_init_body,
        grid=(NBLK,),
        in_specs=[
            pl.BlockSpec((BN_ROWS, 1), lambda i: (i, 0)),
            pl.BlockSpec((BN_ROWS, 1), lambda i: (i, 0)),
            pl.BlockSpec((BN_ROWS, 1), lambda i: (i, 0)),
            pl.BlockSpec((VPAD, DP), lambda i: (0, 0)),
            pl.BlockSpec((2, DP), lambda i: (0, 0)),
            pl.BlockSpec((1, DP), lambda i: (0, 0)),
            pl.BlockSpec((DP, DP), lambda i: (0, 0)),
            pl.BlockSpec((1, DP), lambda i: (0, 0)),
        ],
        out_specs=pl.BlockSpec((2, BN_ROWS, DH), lambda i: (0, i, 0)),
        out_shape=jax.ShapeDtypeStruct((2, N, DH), jnp.float32),
        compiler_params=pltpu.CompilerParams(
            dimension_semantics=("arbitrary",)),
    )(z2, ch2, fc2, at_p, w1, b1, w2, b2)


def _mlp1_body(hin_ref, w1_ref, b1_ref, w2_ref, b2_ref, h2_ref, st_ref,
               acc_ref):
    i = pl.program_id(0)
    t = (jnp.dot(hin_ref[0], w1_ref[:DH, :], preferred_element_type=jnp.float32)
         + jnp.dot(hin_ref[1], w1_ref[DH:, :], preferred_element_type=jnp.float32)
         + b1_ref[:])
    t = jnp.maximum(t, 0.0)
    h2 = jnp.dot(t, w2_ref[:], preferred_element_type=jnp.float32) + b2_ref[:]
    h2_ref[0] = h2[:, :DH]
    h2_ref[1] = h2[:, DH:]
    ssum = jnp.sum(h2, axis=0, keepdims=True)
    ssq = jnp.sum(h2 * h2, axis=0, keepdims=True)
    blk = jnp.concatenate([ssum, ssq], axis=0)

    @pl.when(i == 0)
    def _():
        acc_ref[:] = blk

    @pl.when(i > 0)
    def _():
        acc_ref[:] = acc_ref[:] + blk

    @pl.when(i == NBLK - 1)
    def _():
        st_ref[:] = acc_ref[:]


def _mlp2_body(h2_ref, st_ref, g_ref, bt_ref, out_ref):
    m = st_ref[0:1, :] * (1.0 / N)
    v = st_ref[1:2, :] * (1.0 / N) - m * m
    inv = lax.rsqrt(v + 1e-5)
    for half in range(2):
        lo, hi = half * DH, (half + 1) * DH
        hn = ((h2_ref[half] - m[:, lo:hi]) * inv[:, lo:hi] * g_ref[:, lo:hi]
              + bt_ref[:, lo:hi])
        out_ref[half] = jnp.maximum(hn, 0.0)


def _mlp_bn(hin, w1, b1, w2, b2, g, bt):
    h2, st = pl.pallas_call(
        _mlp1_body,
        grid=(NBLK,),
        in_specs=[
            pl.BlockSpec((2, BN_ROWS, DH), lambda i: (0, i, 0)),
            pl.BlockSpec((DP, DP), lambda i: (0, 0)),
            pl.BlockSpec((1, DP), lambda i: (0, 0)),
            pl.BlockSpec((DP, DP), lambda i: (0, 0)),
            pl.BlockSpec((1, DP), lambda i: (0, 0)),
        ],
        out_specs=[
            pl.BlockSpec((2, BN_ROWS, DH), lambda i: (0, i, 0)),
            pl.BlockSpec((2, DP), lambda i: (0, 0)),
        ],
        out_shape=[
            jax.ShapeDtypeStruct((2, N, DH), jnp.float32),
            jax.ShapeDtypeStruct((2, DP), jnp.float32),
        ],
        scratch_shapes=[pltpu.VMEM((2, DP), jnp.float32)],
        compiler_params=pltpu.CompilerParams(
            dimension_semantics=("arbitrary",)),
    )(hin, w1, b1, w2, b2)
    return pl.pallas_call(
        _mlp2_body,
        grid=(NBLK,),
        in_specs=[
            pl.BlockSpec((2, BN_ROWS, DH), lambda i: (0, i, 0)),
            pl.BlockSpec((2, DP), lambda i: (0, 0)),
            pl.BlockSpec((1, DP), lambda i: (0, 0)),
            pl.BlockSpec((1, DP), lambda i: (0, 0)),
        ],
        out_specs=pl.BlockSpec((2, BN_ROWS, DH), lambda i: (0, i, 0)),
        out_shape=jax.ShapeDtypeStruct((2, N, DH), jnp.float32),
        compiler_params=pltpu.CompilerParams(
            dimension_semantics=("arbitrary",)),
    )(h2, st, g, bt)


def _pool_body(h_ref, b_ref, w_ref, bias_ref, out_ref):
    h = jnp.concatenate([h_ref[0], h_ref[1]], axis=1)
    onehot_t = jnp.where(
        lax.broadcasted_iota(jnp.int32, (NGRAPH, N), 0) == b_ref[:], 1.0, 0.0)
    sums = jnp.dot(onehot_t, h, preferred_element_type=jnp.float32)
    counts = jnp.sum(onehot_t, axis=1, keepdims=True)
    pooled = jnp.where(counts > 0, sums / jnp.maximum(counts, 1.0), 0.0)
    out_ref[:] = (jnp.dot(pooled, w_ref[:], preferred_element_type=jnp.float32)
                  + bias_ref[:])


def _pool(h, b2, w, bias):
    return pl.pallas_call(
        _pool_body,
        out_shape=jax.ShapeDtypeStruct((NGRAPH, DP), jnp.float32),
    )(h, b2, w, bias)


def _padc(x, cols=DP):
    return jnp.pad(x, ((0, 0), (0, cols - x.shape[1])))


def kernel(z, chirality, formal_charge, edge_index, edge_attr, batch,
           atom_table, nap_w1, nap_b1, nap_w2, nap_b2, ee_w1, ee_b1,
           ee_w2, ee_b2, mlp_w1, mlp_b1, mlp_w2, mlp_b2, bn_gamma, bn_beta,
           pool_w, pool_b):
    f32 = jnp.float32
    at_p = jnp.pad(atom_table.astype(f32), ((0, VPAD - atom_table.shape[0]),
                                            (0, DP - D)))
    nap_w1p = _padc(nap_w1)
    nap_b1p = _padc(nap_b1[None, :])
    nap_w2p = _padc(jnp.pad(nap_w2, ((0, DP - D), (0, 0))))
    nap_b2p = _padc(nap_b2[None, :])
    ee_w1p = _padc(ee_w1)
    ee_b1p = _padc(ee_b1[None, :])
    ee_w2p = _padc(jnp.pad(ee_w2, ((0, DP - D), (0, 0))))
    ee_b2p = _padc(ee_b2[None, :])
    w1p = jnp.pad(mlp_w1, ((0, 0), (0, DP - D), (0, DP - D)))
    w2p = jnp.pad(mlp_w2, ((0, 0), (0, DP - D), (0, DP - D)))
    b1p = jnp.pad(mlp_b1, ((0, 0), (0, DP - D)))[:, None, :]
    b2p = jnp.pad(mlp_b2, ((0, 0), (0, DP - D)))[:, None, :]
    gp = jnp.pad(bn_gamma, ((0, 0), (0, DP - D)))[:, None, :]
    btp = jnp.pad(bn_beta, ((0, 0), (0, DP - D)))[:, None, :]
    pool_wp = _padc(jnp.pad(pool_w, ((0, DP - D), (0, 0))))
    pool_bp = _padc(pool_b[None, :])

    src = edge_index[0].astype(jnp.int32)
    dst = edge_index[1].astype(jnp.int32)
    srcx = jnp.concatenate([src, src + N])
    z2 = z.astype(jnp.int32)[:, None]
    ch2 = chirality.astype(f32)[:, None]
    fc2 = formal_charge.astype(f32)[:, None]
    b2d = batch.astype(jnp.int32)[None, :]

    ee = _ee_mlp(edge_attr.astype(f32), ee_w1p, ee_b1p, ee_w2p, ee_b2p)
    h = _node_init(z2, ch2, fc2, at_p, nap_w1p, nap_b1p, nap_w2p, nap_b2p)
    ee_flat = ee.reshape(2 * E, DH)
    for i in range(L):
        hin = _edge_sc(h.reshape(2 * N, DH), ee_flat, srcx, dst)
        h = _mlp_bn(hin.reshape(2, N, DH), w1p[i], b1p[i], w2p[i], b2p[i],
                    gp[i], btp[i])
    out = _pool(h, b2d, pool_wp, pool_bp)
    return out[:, :D]
